# Initial kernel scaffold; baseline (speedup 1.0000x reference)
import jax
import jax.numpy as jnp
from jax.experimental import pallas as pl


def kernel(positions, cell, first_atom_idx, second_atom_idx, shift_vectors, atom_types, W_spec, b_spec, Wr0, br0, Wr1, br1, Wr2, br2, Wr3, br3, Wa0, ba0, Wa1, ba1, Wa2, ba2):
    raise NotImplementedError("write your pallas kernel here")


# speedup vs baseline: 1.4676x; 1.4676x over previous
import functools
import numpy as np
import jax
import jax.numpy as jnp
from jax import lax
from jax.experimental import pallas as pl

N = 10000
E = 160000
NRAD = 8
THETAN = 4
NSPEC = 2
SPEC = NSPEC * NSPEC
ZETA = 2.0
RCUT = 5.0
MAXN = 40
FEAT = (NRAD + NRAD * THETAN) * SPEC

BE = 1000
BN = 8
BM = 1000

_f32 = jnp.float32


def _silu(x):
    return x * (1.0 / (1.0 + jnp.exp(-x)))


def _edge_body(pi_ref, pj_ref, sh_ref, si_ref, sj_ref, cell_ref,
               w0_ref, b0_ref, w1_ref, b1_ref, w2_ref, b2_ref, w3_ref, b3_ref,
               rad_ref, pay_ref):
    pi = pi_ref[...]
    pj = pj_ref[...]
    sh = sh_ref[...]
    shc = (sh[:, 0:1] * cell_ref[0:1, :] + sh[:, 1:2] * cell_ref[1:2, :]
           + sh[:, 2:3] * cell_ref[2:3, :])
    rij = pj - pi + shc
    d = jnp.sqrt(jnp.sum(rij * rij, axis=1, keepdims=True) + 1e-16)
    fc = 0.5 * (jnp.cos(jnp.pi * d / RCUT) + 1.0) * (d < RCUT).astype(_f32)
    rs = lax.broadcasted_iota(jnp.int32, (1, NRAD), 1).astype(_f32) * ((RCUT - 0.5) / (NRAD - 1)) + 0.5
    g = jnp.exp(-4.0 * (d - rs) ** 2)
    h = _silu(jnp.dot(g, w0_ref[...], preferred_element_type=_f32) + b0_ref[...])
    h = _silu(jnp.dot(h, w1_ref[...], preferred_element_type=_f32) + b1_ref[...])
    h = _silu(jnp.dot(h, w2_ref[...], preferred_element_type=_f32) + b2_ref[...])
    r_out = (jnp.dot(h, w3_ref[...], preferred_element_type=_f32) + b3_ref[...]) * fc
    si = si_ref[...]
    sj = sj_ref[...]
    spec = jnp.concatenate([si[:, 0:1] * sj[:, 0:1], si[:, 0:1] * sj[:, 1:2],
                            si[:, 1:2] * sj[:, 0:1], si[:, 1:2] * sj[:, 1:2]], axis=1)
    rad_ref[...] = jnp.concatenate([r_out[:, r:r + 1] * spec for r in range(NRAD)], axis=1)
    rhat = rij / (d + 1e-8)
    gfc = g * fc
    zero = jnp.zeros_like(d)
    pay_ref[...] = jnp.concatenate([rhat, gfc, spec, zero], axis=1)


def _atom_body(pay_ref, radf_ref, feat_ref):
    p = pay_ref[...]
    rhat = p[:, :, 0:3]
    gfc = p[:, :, 3:11]
    spec = p[:, :, 11:15]
    h = jnp.concatenate([gfc[:, :, r:r + 1] * spec for r in range(NRAD)], axis=2)
    rflat = jnp.reshape(rhat, (BN * MAXN, 3))
    cmat = lax.dot_general(rflat, rflat, (((1,), (1,)), ((), ())),
                           preferred_element_type=_f32)
    cmat = jnp.clip(cmat, -1.0, 1.0)
    rowi = lax.broadcasted_iota(jnp.int32, (BN * MAXN, BN * MAXN), 0)
    coli = lax.broadcasted_iota(jnp.int32, (BN * MAXN, BN * MAXN), 1)
    mask = ((rowi // MAXN == coli // MAXN) & (rowi != coli)).astype(_f32)
    smat = jnp.sqrt(jnp.maximum(1.0 - cmat * cmat, 1e-12))
    hflat = jnp.reshape(h, (BN * MAXN, NRAD * SPEC))
    pieces = [radf_ref[...]]
    for t in range(THETAN):
        th = float(t + 1) * np.pi / THETAN
        ct, st = float(np.cos(th)), float(np.sin(th))
        base = 1.0 + cmat * ct + smat * st
        w = (0.5 * base * base) * mask
        y = jnp.reshape(jnp.dot(w, hflat, preferred_element_type=_f32),
                        (BN, MAXN, NRAD * SPEC))
        pieces.append(jnp.sum(h * y, axis=1))
    feat_ref[...] = jnp.concatenate(pieces, axis=1)


def _mlp_body(f_ref, w0_ref, b0_ref, w1_ref, b1_ref, w2_ref, e_ref):
    a = jnp.tanh(jnp.dot(f_ref[...], w0_ref[...], preferred_element_type=_f32) + b0_ref[...])
    a = jnp.tanh(jnp.dot(a, w1_ref[...], preferred_element_type=_f32) + b1_ref[...])
    e_ref[...] = jnp.dot(a, w2_ref[...], preferred_element_type=_f32)


def kernel(positions, cell, first_atom_idx, second_atom_idx, shift_vectors, atom_types,
           W_spec, b_spec, Wr0, br0, Wr1, br1, Wr2, br2, Wr3, br3,
           Wa0, ba0, Wa1, ba1, Wa2, ba2):
    first = first_atom_idx
    second = second_atom_idx
    senc = jnp.take(W_spec, atom_types, axis=0) + b_spec[None, :]
    pi = jnp.take(positions, first, axis=0)
    pj = jnp.take(positions, second, axis=0)
    si = jnp.take(senc, first, axis=0)
    sj = jnp.take(senc, second, axis=0)
    cellp = jnp.pad(cell, ((0, 5), (0, 0)))

    full = lambda shape: pl.BlockSpec(shape, lambda i: (0, 0))
    eb = lambda w: pl.BlockSpec((BE, w), lambda i: (i, 0))
    rad_e, payload = pl.pallas_call(
        _edge_body,
        grid=(E // BE,),
        in_specs=[eb(3), eb(3), eb(3), eb(2), eb(2), full((8, 3)),
                  full((NRAD, 128)), full((1, 128)), full((128, 128)), full((1, 128)),
                  full((128, 128)), full((1, 128)), full((128, NRAD)), full((1, NRAD))],
        out_specs=[eb(NRAD * SPEC), eb(16)],
        out_shape=[jax.ShapeDtypeStruct((E, NRAD * SPEC), _f32),
                   jax.ShapeDtypeStruct((E, 16), _f32)],
    )(pi, pj, shift_vectors, si, sj, cellp,
      Wr0, br0.reshape(1, 128), Wr1, br1.reshape(1, 128),
      Wr2, br2.reshape(1, 128), Wr3, br3.reshape(1, NRAD))

    rad_feat = jax.ops.segment_sum(rad_e, first, num_segments=N)
    counts = jnp.bincount(first, length=N)
    starts = jnp.concatenate([jnp.zeros((1,), counts.dtype), jnp.cumsum(counts)[:-1]])
    pos_in = jnp.arange(E, dtype=counts.dtype) - jnp.take(starts, first)
    row = jnp.where(pos_in < MAXN, first, N)
    col = jnp.minimum(pos_in, MAXN - 1)
    pay_dense = jnp.zeros((N + 1, MAXN, 16), _f32).at[row, col].set(payload)[:N]

    feat = pl.pallas_call(
        _atom_body,
        grid=(N // BN,),
        in_specs=[pl.BlockSpec((BN, MAXN, 16), lambda i: (i, 0, 0)),
                  pl.BlockSpec((BN, NRAD * SPEC), lambda i: (i, 0))],
        out_specs=pl.BlockSpec((BN, FEAT), lambda i: (i, 0)),
        out_shape=jax.ShapeDtypeStruct((N, FEAT), _f32),
    )(pay_dense, rad_feat)

    e = pl.pallas_call(
        _mlp_body,
        grid=(N // BM,),
        in_specs=[pl.BlockSpec((BM, FEAT), lambda i: (i, 0)),
                  full((FEAT, 64)), full((1, 64)), full((64, 64)), full((1, 64)),
                  full((64, 1))],
        out_specs=pl.BlockSpec((BM, 1), lambda i: (i, 0)),
        out_shape=jax.ShapeDtypeStruct((N, 1), _f32),
    )(feat, Wa0, ba0.reshape(1, 64), Wa1, ba1.reshape(1, 64), Wa2)

    return e[:, 0] + ba2[0]
